# baseline (speedup 1.0000x reference)
import functools

import jax
import jax.numpy as jnp
from jax import lax
from jax.experimental import pallas as pl
from jax.experimental.pallas import tpu as pltpu
from jax.experimental.pallas import tpu_sc as plsc

_EPS = 1e-5
_CHUNK = 32
_NBUF = 12
_BC = 2048


@functools.cache
def _build_gather(B, D):
    info = plsc.get_sparse_core_info()
    NC, NS = info.num_cores, info.num_subcores
    NW = NC * NS
    b_per_w = B // NW
    n_chunks = b_per_w // _CHUNK
    nbuf = min(_NBUF, n_chunks)
    mesh = plsc.VectorSubcoreMesh(core_axis_name="c", subcore_axis_name="s")

    @functools.partial(
        pl.kernel,
        mesh=mesh,
        out_type=jax.ShapeDtypeStruct((B, D), jnp.float32),
        scratch_types=[
            pltpu.VMEM((b_per_w,), jnp.int32),
            pltpu.VMEM((nbuf, _CHUNK, D), jnp.float32),
            pltpu.SemaphoreType.DMA,
            pltpu.SemaphoreType.DMA,
        ],
    )
    def gather_k(table_hbm, idx_hbm, out_hbm, idx_v, rows_v, gsem, wsem):
        wid = lax.axis_index("s") * NC + lax.axis_index("c")
        base = wid * b_per_w
        pltpu.sync_copy(idx_hbm.at[pl.ds(base, b_per_w)], idx_v)

        def start_gather(c):
            return pltpu.async_copy(
                table_hbm.at[idx_v.at[pl.ds(c * _CHUNK, _CHUNK)]],
                rows_v.at[c % nbuf], gsem)

        gathers = [start_gather(c) for c in range(nbuf)]
        writes = []
        for c in range(n_chunks):
            gathers[c].wait()
            writes.append(pltpu.async_copy(
                rows_v.at[c % nbuf],
                out_hbm.at[pl.ds(base + c * _CHUNK, _CHUNK)], wsem))
            if c + nbuf < n_chunks:
                writes[c].wait()
                gathers.append(start_gather(c + nbuf))
        for c in range(max(n_chunks - nbuf, 0), n_chunks):
            writes[c].wait()

    return gather_k


def _dense_body(emb_ref, w_ref, g_ref, b_ref, out_ref, h_ref, stats_ref):
    p = pl.program_id(0)
    i = pl.program_id(1)

    @pl.when(p == 0)
    def _matmul_phase():
        hc = lax.dot_general(
            emb_ref[...], w_ref[...], (((1,), (1,)), ((), ())),
            preferred_element_type=jnp.float32,
        )
        hc = jnp.maximum(hc, 0.0)
        h_ref[pl.ds(i * _BC, _BC), :] = hc
        part = jnp.concatenate(
            [jnp.sum(hc, axis=0, keepdims=True),
             jnp.sum(hc * hc, axis=0, keepdims=True)], axis=0)

        @pl.when(i == 0)
        def _():
            stats_ref[...] = part

        @pl.when(i > 0)
        def _():
            stats_ref[...] = stats_ref[...] + part

    @pl.when(p == 1)
    def _normalize_phase():
        n_rows = h_ref.shape[0]
        stats = stats_ref[...]
        mean = stats[0:1, :] * (1.0 / n_rows)
        var = stats[1:2, :] * (1.0 / n_rows) - mean * mean
        scale = g_ref[...] * lax.rsqrt(var + _EPS)
        hc = h_ref[pl.ds(i * _BC, _BC), :]
        out_ref[...] = scale * (hc - mean) + b_ref[...]


def kernel(x, table, W, gamma, beta):
    B = x.shape[0]
    D = table.shape[1]
    DOUT = W.shape[0]
    emb = _build_gather(B, D)(table, x.astype(jnp.int32))
    last = B // _BC - 1
    out = pl.pallas_call(
        _dense_body,
        grid=(2, B // _BC),
        in_specs=[
            pl.BlockSpec((_BC, D),
                         lambda p, i: (jnp.where(p == 0, i, last), 0)),
            pl.BlockSpec((DOUT, D), lambda p, i: (0, 0)),
            pl.BlockSpec((1, DOUT), lambda p, i: (0, 0)),
            pl.BlockSpec((1, DOUT), lambda p, i: (0, 0)),
        ],
        out_specs=pl.BlockSpec((_BC, DOUT),
                               lambda p, i: (jnp.where(p == 0, 0, i), 0)),
        out_shape=jax.ShapeDtypeStruct((B, DOUT), jnp.float32),
        scratch_shapes=[
            pltpu.VMEM((B, DOUT), jnp.float32),
            pltpu.VMEM((2, DOUT), jnp.float32),
        ],
    )(emb, W, gamma.reshape(1, -1), beta.reshape(1, -1))
    return out


# speedup vs baseline: 1.0000x; 1.0000x over previous
import functools

import jax
import jax.numpy as jnp
from jax import lax
from jax.experimental import pallas as pl
from jax.experimental.pallas import tpu as pltpu
from jax.experimental.pallas import tpu_sc as plsc

_EPS = 1e-5
_CHUNK = 64
_NBUF = 6
_BC = 2048


@functools.cache
def _build_gather(B, D):
    info = plsc.get_sparse_core_info()
    NC, NS = info.num_cores, info.num_subcores
    NW = NC * NS
    b_per_w = B // NW
    n_chunks = b_per_w // _CHUNK
    nbuf = min(_NBUF, n_chunks)
    mesh = plsc.VectorSubcoreMesh(core_axis_name="c", subcore_axis_name="s")

    @functools.partial(
        pl.kernel,
        mesh=mesh,
        out_type=jax.ShapeDtypeStruct((B, D), jnp.float32),
        scratch_types=[
            pltpu.VMEM((b_per_w,), jnp.int32),
            pltpu.VMEM((nbuf, _CHUNK, D), jnp.float32),
            pltpu.SemaphoreType.DMA,
            pltpu.SemaphoreType.DMA,
        ],
    )
    def gather_k(table_hbm, idx_hbm, out_hbm, idx_v, rows_v, gsem, wsem):
        wid = lax.axis_index("s") * NC + lax.axis_index("c")
        base = wid * b_per_w
        pltpu.sync_copy(idx_hbm.at[pl.ds(base, b_per_w)], idx_v)

        def start_gather(c):
            return pltpu.async_copy(
                table_hbm.at[idx_v.at[pl.ds(c * _CHUNK, _CHUNK)]],
                rows_v.at[c % nbuf], gsem)

        gathers = [start_gather(c) for c in range(nbuf)]
        writes = []
        for c in range(n_chunks):
            gathers[c].wait()
            writes.append(pltpu.async_copy(
                rows_v.at[c % nbuf],
                out_hbm.at[pl.ds(base + c * _CHUNK, _CHUNK)], wsem))
            if c + nbuf < n_chunks:
                writes[c].wait()
                gathers.append(start_gather(c + nbuf))
        for c in range(max(n_chunks - nbuf, 0), n_chunks):
            writes[c].wait()

    return gather_k


def _dense_body(emb_ref, w_ref, g_ref, b_ref, out_ref, h_ref, stats_ref):
    p = pl.program_id(0)
    i = pl.program_id(1)

    @pl.when(p == 0)
    def _matmul_phase():
        hc = lax.dot_general(
            emb_ref[...], w_ref[...], (((1,), (1,)), ((), ())),
            preferred_element_type=jnp.float32,
        )
        hc = jnp.maximum(hc, 0.0)
        h_ref[pl.ds(i * _BC, _BC), :] = hc
        part = jnp.concatenate(
            [jnp.sum(hc, axis=0, keepdims=True),
             jnp.sum(hc * hc, axis=0, keepdims=True)], axis=0)

        @pl.when(i == 0)
        def _():
            stats_ref[...] = part

        @pl.when(i > 0)
        def _():
            stats_ref[...] = stats_ref[...] + part

    @pl.when(p == 1)
    def _normalize_phase():
        n_rows = h_ref.shape[0]
        stats = stats_ref[...]
        mean = stats[0:1, :] * (1.0 / n_rows)
        var = stats[1:2, :] * (1.0 / n_rows) - mean * mean
        scale = g_ref[...] * lax.rsqrt(var + _EPS)
        hc = h_ref[pl.ds(i * _BC, _BC), :]
        out_ref[...] = scale * (hc - mean) + b_ref[...]


def kernel(x, table, W, gamma, beta):
    B = x.shape[0]
    D = table.shape[1]
    DOUT = W.shape[0]
    emb = _build_gather(B, D)(table, x.astype(jnp.int32))
    last = B // _BC - 1
    out = pl.pallas_call(
        _dense_body,
        grid=(2, B // _BC),
        in_specs=[
            pl.BlockSpec((_BC, D),
                         lambda p, i: (jnp.where(p == 0, i, last), 0)),
            pl.BlockSpec((DOUT, D), lambda p, i: (0, 0)),
            pl.BlockSpec((1, DOUT), lambda p, i: (0, 0)),
            pl.BlockSpec((1, DOUT), lambda p, i: (0, 0)),
        ],
        out_specs=pl.BlockSpec((_BC, DOUT),
                               lambda p, i: (jnp.where(p == 0, 0, i), 0)),
        out_shape=jax.ShapeDtypeStruct((B, DOUT), jnp.float32),
        scratch_shapes=[
            pltpu.VMEM((B, DOUT), jnp.float32),
            pltpu.VMEM((2, DOUT), jnp.float32),
        ],
    )(emb, W, gamma.reshape(1, -1), beta.reshape(1, -1))
    return out


# speedup vs baseline: 1.0738x; 1.0738x over previous
import functools

import jax
import jax.numpy as jnp
from jax import lax
from jax.experimental import pallas as pl
from jax.experimental.pallas import tpu as pltpu
from jax.experimental.pallas import tpu_sc as plsc

_EPS = 1e-5
_CHUNK = 64
_NBUF = 6
_BC = 4096


@functools.cache
def _build_gather(B, D):
    info = plsc.get_sparse_core_info()
    NC, NS = info.num_cores, info.num_subcores
    NW = NC * NS
    b_per_w = B // NW
    n_chunks = b_per_w // _CHUNK
    nbuf = min(_NBUF, n_chunks)
    mesh = plsc.VectorSubcoreMesh(core_axis_name="c", subcore_axis_name="s")

    @functools.partial(
        pl.kernel,
        mesh=mesh,
        out_type=jax.ShapeDtypeStruct((B, D), jnp.float32),
        scratch_types=[
            pltpu.VMEM((b_per_w,), jnp.int32),
            pltpu.VMEM((nbuf, _CHUNK, D), jnp.float32),
            pltpu.SemaphoreType.DMA,
            pltpu.SemaphoreType.DMA,
        ],
    )
    def gather_k(table_hbm, idx_hbm, out_hbm, idx_v, rows_v, gsem, wsem):
        wid = lax.axis_index("s") * NC + lax.axis_index("c")
        base = wid * b_per_w
        pltpu.sync_copy(idx_hbm.at[pl.ds(base, b_per_w)], idx_v)

        def start_gather(c):
            return pltpu.async_copy(
                table_hbm.at[idx_v.at[pl.ds(c * _CHUNK, _CHUNK)]],
                rows_v.at[c % nbuf], gsem)

        gathers = [start_gather(c) for c in range(nbuf)]
        writes = []
        for c in range(n_chunks):
            gathers[c].wait()
            writes.append(pltpu.async_copy(
                rows_v.at[c % nbuf],
                out_hbm.at[pl.ds(base + c * _CHUNK, _CHUNK)], wsem))
            if c + nbuf < n_chunks:
                writes[c].wait()
                gathers.append(start_gather(c + nbuf))
        for c in range(max(n_chunks - nbuf, 0), n_chunks):
            writes[c].wait()

    return gather_k


def _dense_body(emb_ref, w_ref, g_ref, b_ref, out_ref, h_ref, stats_ref):
    p = pl.program_id(0)
    i = pl.program_id(1)

    @pl.when(p == 0)
    def _matmul_phase():
        hc = lax.dot_general(
            emb_ref[...], w_ref[...], (((1,), (1,)), ((), ())),
            preferred_element_type=jnp.float32,
        )
        hc = jnp.maximum(hc, 0.0)
        h_ref[pl.ds(i * _BC, _BC), :] = hc
        part = jnp.concatenate(
            [jnp.sum(hc, axis=0, keepdims=True),
             jnp.sum(hc * hc, axis=0, keepdims=True)], axis=0)

        @pl.when(i == 0)
        def _():
            stats_ref[...] = part

        @pl.when(i > 0)
        def _():
            stats_ref[...] = stats_ref[...] + part

    @pl.when(p == 1)
    def _normalize_phase():
        n_rows = h_ref.shape[0]
        stats = stats_ref[...]
        mean = stats[0:1, :] * (1.0 / n_rows)
        var = stats[1:2, :] * (1.0 / n_rows) - mean * mean
        scale = g_ref[...] * lax.rsqrt(var + _EPS)
        hc = h_ref[pl.ds(i * _BC, _BC), :]
        out_ref[...] = scale * (hc - mean) + b_ref[...]


def kernel(x, table, W, gamma, beta):
    B = x.shape[0]
    D = table.shape[1]
    DOUT = W.shape[0]
    emb = _build_gather(B, D)(table, x.astype(jnp.int32))
    last = B // _BC - 1
    out = pl.pallas_call(
        _dense_body,
        grid=(2, B // _BC),
        in_specs=[
            pl.BlockSpec((_BC, D),
                         lambda p, i: (jnp.where(p == 0, i, last), 0)),
            pl.BlockSpec((DOUT, D), lambda p, i: (0, 0)),
            pl.BlockSpec((1, DOUT), lambda p, i: (0, 0)),
            pl.BlockSpec((1, DOUT), lambda p, i: (0, 0)),
        ],
        out_specs=pl.BlockSpec((_BC, DOUT),
                               lambda p, i: (jnp.where(p == 0, 0, i), 0)),
        out_shape=jax.ShapeDtypeStruct((B, DOUT), jnp.float32),
        scratch_shapes=[
            pltpu.VMEM((B, DOUT), jnp.float32),
            pltpu.VMEM((2, DOUT), jnp.float32),
        ],
    )(emb, W, gamma.reshape(1, -1), beta.reshape(1, -1))
    return out
